# baseline (speedup 1.0000x reference)
import functools

import jax
import jax.numpy as jnp
from jax import lax
from jax.experimental import pallas as pl
from jax.experimental.pallas import tpu as pltpu
from jax.experimental.pallas import tpu_sc as plsc

VOCAB = 100000
D = 300
MAINW = 256
TAILW = 128
TOFF = 172
OV = MAINW - TOFF
B = 4096 * 50
CHUNK = 128
NW = 32
CPT = B // (NW * CHUNK)


def _make_gather():
    mesh = plsc.VectorSubcoreMesh(core_axis_name="c", subcore_axis_name="s")

    @functools.partial(
        pl.kernel,
        mesh=mesh,
        out_type=jax.ShapeDtypeStruct((B, D), jnp.float32),
        compiler_params=pltpu.CompilerParams(needs_layout_passes=False),
        scratch_types=[
            pltpu.VMEM((CPT, CHUNK), jnp.int32),
            pltpu.VMEM((CHUNK, D), jnp.float32),
            pltpu.VMEM((CHUNK, D), jnp.float32),
            pltpu.VMEM((CHUNK, TAILW), jnp.float32),
            pltpu.SemaphoreType.DMA,
            pltpu.SemaphoreType.DMA,
            pltpu.SemaphoreType.DMA,
            pltpu.SemaphoreType.DMA,
            pltpu.SemaphoreType.DMA,
        ],
    )
    def gather(idx_hbm, table_hbm, tail_hbm, out_hbm,
               idx_v, o0, o1, b, sa0, sa1, sb, sw0, sw1):
        wid = lax.axis_index("s") * 2 + lax.axis_index("c")
        base_chunk = wid * CPT
        tmain = table_hbm.at[:, pl.ds(0, MAINW)]
        pltpu.sync_copy(idx_hbm.at[wid], idx_v)

        def start_main(c, o, sa):
            pltpu.async_copy(tmain.at[idx_v.at[c]], o.at[:, pl.ds(0, MAINW)], sa)

        def wait_main(c, o, sa):
            pltpu.make_async_copy(
                tmain.at[idx_v.at[c]], o.at[:, pl.ds(0, MAINW)], sa).wait()

        def start_tail(c):
            pltpu.async_copy(tail_hbm.at[idx_v.at[c]], b, sb)

        def wait_tail(c):
            pltpu.make_async_copy(tail_hbm.at[idx_v.at[c]], b, sb).wait()

        def stitch(b, o):
            lane = lax.iota(jnp.int32, 16)
            cols = MAINW + 28 + lane
            msk = cols >= MAINW + 32

            def row(r, _):
                o[r, pl.ds(MAINW, 16)] = b[r, pl.ds(OV, 16)]
                o[r, pl.ds(MAINW + 16, 16)] = b[r, pl.ds(OV + 16, 16)]
                v = b[r, pl.ds(OV + 28, 16)]
                rows16 = jnp.full((16,), r, jnp.int32)
                plsc.store_scatter(o, [rows16, cols], v, mask=msk)
                return ()
            lax.fori_loop(0, CHUNK, row, ())

        def wait_write(o, sw):
            pltpu.make_async_copy(o, out_hbm.at[pl.ds(0, CHUNK)], sw).wait()

        start_main(0, o0, sa0)
        start_tail(0)

        def pair(i, _):
            c = i * 2
            wait_main(c, o0, sa0)
            wait_tail(c)

            @pl.when(i > 0)
            def _():
                wait_write(o1, sw1)

            start_main(c + 1, o1, sa1)
            stitch(b, o0)
            start_tail(c + 1)
            pltpu.async_copy(
                o0, out_hbm.at[pl.ds((base_chunk + c) * CHUNK, CHUNK)], sw0)
            wait_main(c + 1, o1, sa1)
            wait_tail(c + 1)

            @pl.when(c + 2 < CPT)
            def _():
                wait_write(o0, sw0)
                start_main(c + 2, o0, sa0)

            stitch(b, o1)

            @pl.when(c + 2 < CPT)
            def _():
                start_tail(c + 2)

            pltpu.async_copy(
                o1, out_hbm.at[pl.ds((base_chunk + c + 1) * CHUNK, CHUNK)], sw1)
            return ()

        lax.fori_loop(0, CPT // 2, pair, ())
        wait_write(o0, sw0)
        wait_write(o1, sw1)

    return gather


_gather = _make_gather()


def kernel(tokens, table):
    idx = tokens.astype(jnp.int32).reshape(NW, CPT, CHUNK)
    msk = jax.lax.optimization_barrier(jnp.ones((1, TAILW), jnp.bool_))
    tail = jnp.where(msk, table[:, TOFF:TOFF + TAILW], 0.0)
    main = jnp.where(msk[:, :1], table, 0.0)
    out = _gather(idx, main, tail)
    return out.reshape(tokens.shape[0], tokens.shape[1], D)


# speedup vs baseline: 1.2390x; 1.2390x over previous
import functools

import jax
import jax.numpy as jnp
from jax import lax
from jax.experimental import pallas as pl
from jax.experimental.pallas import tpu as pltpu
from jax.experimental.pallas import tpu_sc as plsc

VOCAB = 100000
D = 300
MAINW = 256
TAILW = 128
TOFF = 172
OV = MAINW - TOFF
B = 4096 * 50
CHUNK = 128
NW = 32
CPT = B // (NW * CHUNK)


def _make_gather():
    mesh = plsc.VectorSubcoreMesh(core_axis_name="c", subcore_axis_name="s")

    @functools.partial(
        pl.kernel,
        mesh=mesh,
        out_type=jax.ShapeDtypeStruct((B, D), jnp.float32),
        compiler_params=pltpu.CompilerParams(needs_layout_passes=False),
        scratch_types=[
            pltpu.VMEM((CPT, CHUNK), jnp.int32),
            pltpu.VMEM((CHUNK, D), jnp.float32),
            pltpu.VMEM((CHUNK, D), jnp.float32),
            pltpu.VMEM((CHUNK, TAILW), jnp.float32),
            pltpu.SemaphoreType.DMA,
            pltpu.SemaphoreType.DMA,
            pltpu.SemaphoreType.DMA,
            pltpu.SemaphoreType.DMA,
            pltpu.SemaphoreType.DMA,
        ],
    )
    def gather(idx_hbm, table_hbm, tail_hbm, out_hbm,
               idx_v, o0, o1, b, sa0, sa1, sb, sw0, sw1):
        wid = lax.axis_index("s") * 2 + lax.axis_index("c")
        base_chunk = wid * CPT
        tmain = table_hbm.at[:, pl.ds(0, MAINW)]
        pltpu.sync_copy(idx_hbm.at[wid], idx_v)

        def start_main(c, o, sa):
            pltpu.async_copy(tmain.at[idx_v.at[c]], o.at[:, pl.ds(0, MAINW)], sa)

        def wait_main(c, o, sa):
            pltpu.make_async_copy(
                tmain.at[idx_v.at[c]], o.at[:, pl.ds(0, MAINW)], sa).wait()

        def start_tail(c):
            pltpu.async_copy(tail_hbm.at[idx_v.at[c]], b, sb)

        def wait_tail(c):
            pltpu.make_async_copy(tail_hbm.at[idx_v.at[c]], b, sb).wait()

        def stitch(o):
            lane = lax.iota(jnp.int32, 16)
            cols = MAINW + 28 + lane
            msk = cols >= MAINW + 32

            def row(r, _):
                o[r, pl.ds(MAINW, 16)] = b[r, pl.ds(OV, 16)]
                o[r, pl.ds(MAINW + 16, 16)] = b[r, pl.ds(OV + 16, 16)]
                v = b[r, pl.ds(OV + 28, 16)]
                rows16 = jnp.full((16,), r, jnp.int32)
                plsc.store_scatter(o, [rows16, cols], v, mask=msk)
                return ()
            lax.fori_loop(0, CHUNK, row, ())

        def wait_write(o, sw):
            pltpu.make_async_copy(o, out_hbm.at[pl.ds(0, CHUNK)], sw).wait()

        start_main(0, o0, sa0)
        start_tail(0)

        def pair(i, _):
            c = i * 2
            wait_main(c, o0, sa0)
            wait_tail(c)

            @pl.when(i > 0)
            def _():
                wait_write(o1, sw1)

            start_main(c + 1, o1, sa1)
            stitch(o0)
            start_tail(c + 1)
            pltpu.async_copy(
                o0, out_hbm.at[pl.ds((base_chunk + c) * CHUNK, CHUNK)], sw0)
            wait_main(c + 1, o1, sa1)
            wait_tail(c + 1)

            @pl.when(c + 2 < CPT)
            def _():
                wait_write(o0, sw0)
                start_main(c + 2, o0, sa0)

            stitch(o1)

            @pl.when(c + 2 < CPT)
            def _():
                start_tail(c + 2)

            pltpu.async_copy(
                o1, out_hbm.at[pl.ds((base_chunk + c + 1) * CHUNK, CHUNK)], sw1)
            return ()

        lax.fori_loop(0, CPT // 2, pair, ())
        wait_write(o0, sw0)
        wait_write(o1, sw1)

    return gather


_gather = _make_gather()


def kernel(tokens, table):
    idx = tokens.astype(jnp.int32).reshape(NW, CPT, CHUNK)
    msk = jax.lax.optimization_barrier(jnp.ones((1, TAILW), jnp.bool_))
    tail = jnp.where(msk, table[:, TOFF:TOFF + TAILW], 0.0)
    out = _gather(idx, table, tail)
    msk3 = jax.lax.optimization_barrier(jnp.ones((1, 1, 1), jnp.bool_))
    return jnp.where(
        msk3, out.reshape(tokens.shape[0], tokens.shape[1], D), 0.0)


# speedup vs baseline: 1.5102x; 1.2189x over previous
import functools

import jax
import jax.numpy as jnp
from jax import lax
from jax.experimental import pallas as pl
from jax.experimental.pallas import tpu as pltpu
from jax.experimental.pallas import tpu_sc as plsc

VOCAB = 100000
D = 300
MAINW = 256
TAILW = 128
TOFF = 172
OV = MAINW - TOFF
B = 4096 * 50
CHUNK = 128
NW = 32
CPT = B // (NW * CHUNK)


def _make_gather():
    mesh = plsc.VectorSubcoreMesh(core_axis_name="c", subcore_axis_name="s")

    @functools.partial(
        pl.kernel,
        mesh=mesh,
        out_type=jax.ShapeDtypeStruct((B, D), jnp.float32),
        compiler_params=pltpu.CompilerParams(needs_layout_passes=False),
        scratch_types=[
            pltpu.VMEM((CPT, CHUNK), jnp.int32),
            pltpu.VMEM((CHUNK, D), jnp.float32),
            pltpu.VMEM((CHUNK, D), jnp.float32),
            pltpu.VMEM((CHUNK, TAILW), jnp.float32),
            pltpu.SemaphoreType.DMA,
            pltpu.SemaphoreType.DMA,
            pltpu.SemaphoreType.DMA,
            pltpu.SemaphoreType.DMA,
            pltpu.SemaphoreType.DMA,
        ],
    )
    def gather(idx_hbm, table_hbm, tail_hbm, out_hbm,
               idx_v, o0, o1, b, sa0, sa1, sb, sw0, sw1):
        wid = lax.axis_index("s") * 2 + lax.axis_index("c")
        base_chunk = wid * CPT
        tmain = table_hbm.at[:, pl.ds(0, MAINW)]
        pltpu.sync_copy(idx_hbm.at[wid], idx_v)

        def start_main(c, o, sa):
            pltpu.async_copy(tmain.at[idx_v.at[c]], o.at[:, pl.ds(0, MAINW)], sa)

        def wait_main(c, o, sa):
            pltpu.make_async_copy(
                tmain.at[idx_v.at[c]], o.at[:, pl.ds(0, MAINW)], sa).wait()

        def start_tail(c):
            pltpu.async_copy(tail_hbm.at[idx_v.at[c]], b, sb)

        def wait_tail(c):
            pltpu.make_async_copy(tail_hbm.at[idx_v.at[c]], b, sb).wait()

        def stitch(o):
            lane = lax.iota(jnp.int32, 16)
            cols = MAINW + 28 + lane
            msk = cols >= MAINW + 32

            def row(r, _):
                o[r, pl.ds(MAINW, 16)] = b[r, pl.ds(OV, 16)]
                o[r, pl.ds(MAINW + 16, 16)] = b[r, pl.ds(OV + 16, 16)]
                v = b[r, pl.ds(OV + 28, 16)]
                rows16 = jnp.full((16,), r, jnp.int32)
                plsc.store_scatter(o, [rows16, cols], v, mask=msk)
                return ()
            lax.fori_loop(0, CHUNK, row, ())

        def wait_write(o, sw):
            pltpu.make_async_copy(o, out_hbm.at[pl.ds(0, CHUNK)], sw).wait()

        start_main(0, o0, sa0)
        start_tail(0)

        def pair(i, _):
            c = i * 2
            wait_main(c, o0, sa0)
            wait_tail(c)

            @pl.when(i > 0)
            def _():
                wait_write(o1, sw1)

            start_main(c + 1, o1, sa1)
            stitch(o0)
            start_tail(c + 1)
            pltpu.async_copy(
                o0, out_hbm.at[pl.ds((base_chunk + c) * CHUNK, CHUNK)], sw0)
            wait_main(c + 1, o1, sa1)
            wait_tail(c + 1)

            @pl.when(c + 2 < CPT)
            def _():
                wait_write(o0, sw0)
                start_main(c + 2, o0, sa0)

            stitch(o1)

            @pl.when(c + 2 < CPT)
            def _():
                start_tail(c + 2)

            pltpu.async_copy(
                o1, out_hbm.at[pl.ds((base_chunk + c + 1) * CHUNK, CHUNK)], sw1)
            return ()

        lax.fori_loop(0, CPT // 2, pair, ())
        wait_write(o0, sw0)
        wait_write(o1, sw1)

    return gather


_gather = _make_gather()


def kernel(tokens, table):
    idx = tokens.astype(jnp.int32).reshape(NW, CPT, CHUNK)
    msk = jax.lax.optimization_barrier(jnp.ones((1, TAILW), jnp.bool_))
    tail = jnp.where(msk, table[:, TOFF:TOFF + TAILW], 0.0)
    out = _gather(idx, table, tail)
    return out.reshape(tokens.shape[0], tokens.shape[1], D)
